# Initial kernel scaffold; baseline (speedup 1.0000x reference)
import jax
import jax.numpy as jnp
from jax.experimental import pallas as pl


def kernel(x, tables, W, b):
    raise NotImplementedError("write your pallas kernel here")


# speedup vs baseline: 53.0319x; 53.0319x over previous
import functools

import jax
import jax.numpy as jnp
from jax import lax
from jax.experimental import pallas as pl
from jax.experimental.pallas import tpu as pltpu
from jax.experimental.pallas import tpu_sc as plsc

B, S, F = 16384, 200, 16
NCAT = 14
VOCAB = 100000
EDIM = 5
BS = B * S

NC, NS = 2, 16
NW = NC * NS
TPT = BS // NW
T = 640
NCHUNK = TPT // T

VSLC = 6400
VSLC_LAST = VOCAB - 15 * VSLC
VCH = 1600


def _kernel_body(x_hbm, tbl_hbm, wcat_hbm, cvec_hbm, out_hbm,
                 proj_sh, tblb, projb, wcat_v, cvec_v,
                 xb, idxb, valb, outb, gsem):
  cid = lax.axis_index("c")
  sid = lax.axis_index("s")
  iota16 = lax.iota(jnp.int32, 16)

  pltpu.sync_copy(wcat_hbm, wcat_v)
  pltpu.sync_copy(cvec_hbm, cvec_v)

  def project(chunks):
    voff = sid * VSLC

    def one(c, coff, n):
      src = (c * VOCAB + coff) * EDIM + voff * EDIM
      pltpu.sync_copy(tbl_hbm.at[pl.ds(src, n * EDIM)],
                      tblb.at[pl.ds(0, n * EDIM)])
      wc = [plsc.load_gather(wcat_v, [iota16 + (c * EDIM + e) * 16])
            for e in range(EDIM)]

      def gbody(g, _):
        b5 = (g * 16 + iota16) * EDIM
        acc = plsc.load_gather(tblb, [b5]) * wc[0]
        for e in range(1, EDIM):
          acc = acc + plsc.load_gather(tblb, [b5 + e]) * wc[e]
        projb[pl.ds(g * 16, 16)] = acc
        return 0

      lax.fori_loop(0, n // 16, gbody, 0)
      dst = c * VOCAB + coff + voff
      pltpu.sync_copy(projb.at[pl.ds(0, n)], proj_sh.at[pl.ds(dst, n)])

    for c in range(NCAT):
      coff = 0
      for n in chunks:
        one(c, coff, n)
        coff += n

  @pl.when(sid < 15)
  def _():
    project([VCH] * (VSLC // VCH))

  @pl.when(sid == 15)
  def _():
    project([VCH] * (VSLC_LAST // VCH) + [VSLC_LAST % VCH])

  plsc.subcore_barrier()

  wid = sid * NC + cid
  base_tok = wid * TPT
  w0 = plsc.load_gather(cvec_v, [iota16])
  w1 = plsc.load_gather(cvec_v, [iota16 + 16])
  bvec = plsc.load_gather(cvec_v, [iota16 + 32])

  def chunk(g, _):
    tok0 = base_tok + g * T
    pltpu.sync_copy(x_hbm.at[pl.ds(tok0 * F, T * F)], xb)

    def rbody(j, _):
      gi = j * (16 * F) + iota16 * F
      for c in range(NCAT):
        v = plsc.load_gather(xb, [gi + (c + 2)])
        idxb[pl.ds(c * T + j * 16, 16)] = v.astype(jnp.int32) + c * VOCAB
      return 0

    lax.fori_loop(0, T // 16, rbody, 0)

    pltpu.async_copy(proj_sh.at[idxb], valb, gsem).wait()

    def abody(j, _):
      gi = j * (16 * F) + iota16 * F
      x0 = plsc.load_gather(xb, [gi])
      x1 = plsc.load_gather(xb, [gi + 1])
      z = x0 * w0 + x1 * w1 + bvec
      for c in range(NCAT):
        z = z + valb[pl.ds(c * T + j * 16, 16)]
      outb[pl.ds(j * 16, 16)] = 1.0 / (1.0 + jnp.exp(-z))
      return 0

    lax.fori_loop(0, T // 16, abody, 0)
    pltpu.sync_copy(outb, out_hbm.at[pl.ds(tok0, T)])
    return 0

  lax.fori_loop(0, NCHUNK, chunk, 0)


@jax.jit
def kernel(x, tables, W, b):
  x_flat = x.reshape(BS * F)
  tbl_flat = tables.reshape(NCAT * VOCAB * EDIM)
  wcat = jnp.broadcast_to(W[2:, 0].reshape(NCAT, EDIM)[:, :, None],
                          (NCAT, EDIM, 16)).reshape(NCAT * EDIM * 16)
  cvec = jnp.broadcast_to(jnp.stack([W[0, 0], W[1, 0], b[0]])[:, None],
                          (3, 16)).reshape(3 * 16)

  mesh = plsc.VectorSubcoreMesh(core_axis_name="c", subcore_axis_name="s")
  run = pl.kernel(
      _kernel_body,
      out_type=jax.ShapeDtypeStruct((BS,), jnp.float32),
      mesh=mesh,
      compiler_params=pltpu.CompilerParams(needs_layout_passes=False),
      scratch_types=[
          pltpu.VMEM_SHARED((NCAT * VOCAB,), jnp.float32),
          pltpu.VMEM((VCH * EDIM,), jnp.float32),
          pltpu.VMEM((VCH,), jnp.float32),
          pltpu.VMEM((NCAT * EDIM * 16,), jnp.float32),
          pltpu.VMEM((3 * 16,), jnp.float32),
          pltpu.VMEM((T * F,), jnp.float32),
          pltpu.VMEM((NCAT * T,), jnp.int32),
          pltpu.VMEM((NCAT * T,), jnp.float32),
          pltpu.VMEM((T,), jnp.float32),
          pltpu.SemaphoreType.DMA,
      ],
  )
  out = run(x_flat, tbl_flat, wcat, cvec)
  return out.reshape(B, S)


# speedup vs baseline: 136.9499x; 2.5824x over previous
import functools

import jax
import jax.numpy as jnp
from jax import lax
from jax.experimental import pallas as pl
from jax.experimental.pallas import tpu as pltpu
from jax.experimental.pallas import tpu_sc as plsc

B, S, F = 16384, 200, 16
NCAT = 14
VOCAB = 100000
EDIM = 5

NC, NS = 2, 16
NW = NC * NS
BPT = B // NW
CB = 128
HT = 4 * CB
NGS = S // 8
NGB = BPT // CB

NVT = VOCAB // 128
VTAIL = VOCAB - NVT * 128
TPS = 49


def _kernel_body(x_hbm, tbl_hbm, tail_hbm, wcat_hbm, cvec_hbm, out_hbm,
                 proj_sh, tblb, projb, tailb, wcat_v, cvec_v,
                 xb, idxb, valb, outb, gsem):
  cid = lax.axis_index("c")
  sid = lax.axis_index("s")
  iota16 = lax.iota(jnp.int32, 16)
  iota14 = iota16 * NCAT

  pltpu.sync_copy(wcat_hbm, wcat_v)
  pltpu.sync_copy(cvec_hbm, cvec_v)

  def vtile(t, _):
    pltpu.sync_copy(tbl_hbm.at[:, :, pl.ds(t * 128, 128)],
                    tblb)
    for c in range(NCAT):
      wc = [plsc.load_gather(wcat_v, [iota16 + (c * EDIM + e) * 16])
            for e in range(EDIM)]
      for g in range(8):
        acc = tblb[0, c, pl.ds(g * 16, 16)] * wc[0]
        for e in range(1, EDIM):
          acc = acc + tblb[e, c, pl.ds(g * 16, 16)] * wc[e]
        plsc.store_scatter(projb, [iota14 + (g * 16 * NCAT + c)], acc)
    pltpu.sync_copy(projb, proj_sh.at[pl.ds(t * (128 * NCAT), 128 * NCAT)])
    return 0

  ntiles = jnp.where(sid == NS - 1, NVT - (NS - 1) * TPS, TPS)
  lax.fori_loop(sid * TPS, sid * TPS + ntiles, vtile, 0)

  @pl.when(sid == NS - 1)
  def _():
    pltpu.sync_copy(tail_hbm, tailb)

    def tgroup(g, _):
      q = g * 16 + iota16
      v = q // NCAT
      c = q % NCAT
      acc = jnp.zeros((16,), jnp.float32)
      for e in range(EDIM):
        tv = plsc.load_gather(tailb, [(c * VTAIL + v) * EDIM + e])
        wv = plsc.load_gather(wcat_v, [(c * EDIM + e) * 16])
        acc = acc + tv * wv
      projb[pl.ds(g * 16, 16)] = acc
      return 0

    lax.fori_loop(0, VTAIL * NCAT // 16, tgroup, 0)
    pltpu.sync_copy(projb.at[pl.ds(0, VTAIL * NCAT)],
                    proj_sh.at[pl.ds(NVT * 128 * NCAT, VTAIL * NCAT)])

  plsc.subcore_barrier()

  wid = sid * NC + cid
  b_base = wid * BPT
  w0 = plsc.load_gather(cvec_v, [iota16])
  w1 = plsc.load_gather(cvec_v, [iota16 + 16])
  bvec = plsc.load_gather(cvec_v, [iota16 + 32])

  def half(s0, b0, h):
    pltpu.sync_copy(x_hbm.at[pl.ds(s0 + h * 4, 4), :, pl.ds(b0, CB)],
                    xb)
    for si in range(4):
      for bg in range(CB // 16):
        base = si * CB + bg * 16
        for c in range(NCAT):
          v = xb[si, c + 2, pl.ds(bg * 16, 16)]
          idxb[pl.ds(c * HT + base, 16)] = v.astype(jnp.int32) * NCAT + c
    pltpu.async_copy(proj_sh.at[idxb], valb, gsem).wait()
    for si in range(4):
      for bg in range(CB // 16):
        base = si * CB + bg * 16
        z = (xb[si, 0, pl.ds(bg * 16, 16)] * w0
             + xb[si, 1, pl.ds(bg * 16, 16)] * w1 + bvec)
        for c in range(NCAT):
          z = z + valb[pl.ds(c * HT + base, 16)]
        outb[h * 4 + si, pl.ds(bg * 16, 16)] = 1.0 / (1.0 + jnp.exp(-z))

  def gs_loop(gs, _):
    s0 = gs * 8

    def gb_loop(gb, _):
      b0 = b_base + gb * CB
      half(s0, b0, 0)
      half(s0, b0, 1)
      pltpu.sync_copy(outb,
                      out_hbm.at[pl.ds(s0, 8), pl.ds(b0, CB)])
      return 0

    lax.fori_loop(0, NGB, gb_loop, 0)
    return 0

  lax.fori_loop(0, NGS, gs_loop, 0)


@jax.jit
def kernel(x, tables, W, b):
  x_t = x.transpose(1, 2, 0)
  tbl_t = tables.transpose(2, 0, 1)
  tail = tables[:, NVT * 128:, :].reshape(NCAT * VTAIL * EDIM)
  wcat = jnp.broadcast_to(W[2:, 0].reshape(NCAT, EDIM)[:, :, None],
                          (NCAT, EDIM, 16)).reshape(NCAT * EDIM * 16)
  cvec = jnp.broadcast_to(jnp.stack([W[0, 0], W[1, 0], b[0]])[:, None],
                          (3, 16)).reshape(3 * 16)

  mesh = plsc.VectorSubcoreMesh(core_axis_name="c", subcore_axis_name="s")
  run = pl.kernel(
      _kernel_body,
      out_type=jax.ShapeDtypeStruct((S, B), jnp.float32),
      mesh=mesh,
      compiler_params=pltpu.CompilerParams(needs_layout_passes=False, use_tc_tiling_on_sc=False),
      scratch_types=[
          pltpu.VMEM_SHARED((NCAT * VOCAB,), jnp.float32),
          pltpu.VMEM((EDIM, NCAT, 128), jnp.float32),
          pltpu.VMEM((128 * NCAT,), jnp.float32),
          pltpu.VMEM((NCAT * VTAIL * EDIM,), jnp.float32),
          pltpu.VMEM((NCAT * EDIM * 16,), jnp.float32),
          pltpu.VMEM((3 * 16,), jnp.float32),
          pltpu.VMEM((4, F, CB), jnp.float32),
          pltpu.VMEM((NCAT * HT,), jnp.int32),
          pltpu.VMEM((NCAT * HT,), jnp.float32),
          pltpu.VMEM((8, CB), jnp.float32),
          pltpu.SemaphoreType.DMA,
      ],
  )
  out_t = run(x_t, tbl_t, tail, wcat, cvec)
  return out_t.T


# speedup vs baseline: 215.3846x; 1.5727x over previous
import functools

import jax
import jax.numpy as jnp
from jax import lax
from jax.experimental import pallas as pl
from jax.experimental.pallas import tpu as pltpu
from jax.experimental.pallas import tpu_sc as plsc

B, S, F = 16384, 200, 16
NCAT = 14
VOCAB = 100000
EDIM = 5

NC, NS = 2, 16
NW = NC * NS
BPT = B // NW
CB = 128
HT = 2 * CB
NCH = (S // 8) * (BPT // CB)
NHALF = NCH * 4

NVT = VOCAB // 128
VTAIL = VOCAB - NVT * 128
TPS = 49


def _kernel_body(x_hbm, t0_hbm, t1_hbm, t2_hbm, t3_hbm, t4_hbm,
                 wcat_hbm, cvec_hbm, out_hbm,
                 proj_sh, tblb, projb, wcat_v, cvec_v,
                 xb0, xb1, idxb0, idxb1, valb0, valb1, zb0, zb1, outb,
                 sx0, sx1, sg0, sg1, so):
  cid = lax.axis_index("c")
  sid = lax.axis_index("s")
  iota16 = lax.iota(jnp.int32, 16)
  iota14 = iota16 * NCAT
  tbl_hbms = [t0_hbm, t1_hbm, t2_hbm, t3_hbm, t4_hbm]

  pltpu.sync_copy(wcat_hbm, wcat_v)
  pltpu.sync_copy(cvec_hbm, cvec_v)

  def vtile(t, _):
    for e in range(EDIM):
      pltpu.sync_copy(tbl_hbms[e].at[:, pl.ds(t * 128, 128)], tblb.at[e])
    for c in range(NCAT):
      wc = [plsc.load_gather(wcat_v, [iota16 + (c * EDIM + e) * 16])
            for e in range(EDIM)]
      for g in range(8):
        acc = tblb[0, c, pl.ds(g * 16, 16)] * wc[0]
        for e in range(1, EDIM):
          acc = acc + tblb[e, c, pl.ds(g * 16, 16)] * wc[e]
        plsc.store_scatter(projb, [iota14 + (g * 16 * NCAT + c)], acc)
    pltpu.sync_copy(projb, proj_sh.at[pl.ds(t * (128 * NCAT), 128 * NCAT)])
    return 0

  ntiles = jnp.where(sid == NS - 1, NVT - (NS - 1) * TPS, TPS)
  lax.fori_loop(sid * TPS, sid * TPS + ntiles, vtile, 0)

  @pl.when(sid == NS - 1)
  def _():
    for e in range(EDIM):
      pltpu.sync_copy(tbl_hbms[e].at[:, pl.ds(NVT * 128, VTAIL)],
                      tblb.at[e, :, pl.ds(0, VTAIL)])

    def tgroup(g, _):
      q = g * 16 + iota16
      v = q // NCAT
      c = q % NCAT
      acc = jnp.zeros((16,), jnp.float32)
      for e in range(EDIM):
        tv = plsc.load_gather(tblb, [jnp.full((16,), e, jnp.int32), c, v])
        wv = plsc.load_gather(wcat_v, [(c * EDIM + e) * 16])
        acc = acc + tv * wv
      projb[pl.ds(g * 16, 16)] = acc
      return 0

    lax.fori_loop(0, VTAIL * NCAT // 16, tgroup, 0)
    pltpu.sync_copy(projb.at[pl.ds(0, VTAIL * NCAT)],
                    proj_sh.at[pl.ds(NVT * 128 * NCAT, VTAIL * NCAT)])

  plsc.subcore_barrier()

  wid = sid * NC + cid
  b_base = wid * BPT
  w0 = plsc.load_gather(cvec_v, [iota16])
  w1 = plsc.load_gather(cvec_v, [iota16 + 16])
  bvec = plsc.load_gather(cvec_v, [iota16 + 32])

  xbs, idxbs = [xb0, xb1], [idxb0, idxb1]
  valbs, zbs = [valb0, valb1], [zb0, zb1]
  sxs, sgs = [sx0, sx1], [sg0, sg1]

  def xslice(i):
    s0 = (i // 16) * 8 + (i % 4) * 2
    b0 = b_base + ((i // 4) % 4) * CB
    return x_hbm.at[pl.ds(s0, 2), :, pl.ds(b0, CB)]

  def build(i, p):
    xb, idxb, zb = xbs[p], idxbs[p], zbs[p]
    for r in range(2):
      for bg in range(8):
        base = r * CB + bg * 16
        for c in range(NCAT):
          v = xb[r, c + 2, pl.ds(bg * 16, 16)]
          idxb[pl.ds(c * HT + base, 16)] = v.astype(jnp.int32) * NCAT + c
        zb[pl.ds(base, 16)] = (xb[r, 0, pl.ds(bg * 16, 16)] * w0
                               + xb[r, 1, pl.ds(bg * 16, 16)] * w1 + bvec)

  def accum(pj, hp):
    valb, zb = valbs[pj], zbs[pj]
    pltpu.make_async_copy(proj_sh.at[idxbs[pj]], valb, sgs[pj]).wait()
    for r in range(2):
      for bg in range(8):
        base = r * CB + bg * 16
        z = zb[pl.ds(base, 16)]
        for c in range(NCAT):
          z = z + valb[pl.ds(c * HT + base, 16)]
        outb[2 * hp + r, pl.ds(bg * 16, 16)] = 1.0 / (1.0 + jnp.exp(-z))

  def outdma(c):
    return pltpu.make_async_copy(
        outb, out_hbm.at[pl.ds((c // 4) * 8, 8),
                         pl.ds(b_base + (c % 4) * CB, CB)], so)

  pltpu.async_copy(xslice(0), xbs[0], sxs[0])

  def chunk_loop(c, _):
    for h in range(4):
      p = h % 2
      i = c * 4 + h
      pltpu.make_async_copy(xslice(i), xbs[p], sxs[p]).wait()

      @pl.when(i < NHALF - 1)
      def _():
        pltpu.async_copy(xslice(i + 1), xbs[1 - p], sxs[1 - p])

      build(i, p)
      pltpu.async_copy(proj_sh.at[idxbs[p]], valbs[p], sgs[p])
      if h == 0:
        @pl.when(c > 0)
        def _():
          accum(1, 3)
          outdma(c - 1).start()
      elif h == 1:
        @pl.when(c > 0)
        def _():
          outdma(c - 1).wait()
        accum(0, 0)
      else:
        accum((h - 1) % 2, h - 1)
    return 0

  lax.fori_loop(0, NCH, chunk_loop, 0)
  accum(1, 3)
  d = outdma(NCH - 1)
  d.start()
  d.wait()


@jax.jit
def kernel(x, tables, W, b):
  x_t = x.transpose(1, 2, 0)
  tbls = [tables[:, :, e] for e in range(EDIM)]
  wcat = jnp.broadcast_to(W[2:, 0].reshape(NCAT, EDIM)[:, :, None],
                          (NCAT, EDIM, 16)).reshape(NCAT * EDIM * 16)
  cvec = jnp.broadcast_to(jnp.stack([W[0, 0], W[1, 0], b[0]])[:, None],
                          (3, 16)).reshape(3 * 16)

  mesh = plsc.VectorSubcoreMesh(core_axis_name="c", subcore_axis_name="s")
  run = pl.kernel(
      _kernel_body,
      out_type=jax.ShapeDtypeStruct((S, B), jnp.float32),
      mesh=mesh,
      compiler_params=pltpu.CompilerParams(
          needs_layout_passes=False, use_tc_tiling_on_sc=False),
      scratch_types=[
          pltpu.VMEM_SHARED((NCAT * VOCAB,), jnp.float32),
          pltpu.VMEM((EDIM, NCAT, 128), jnp.float32),
          pltpu.VMEM((128 * NCAT,), jnp.float32),
          pltpu.VMEM((NCAT * EDIM * 16,), jnp.float32),
          pltpu.VMEM((3 * 16,), jnp.float32),
          pltpu.VMEM((2, F, CB), jnp.float32),
          pltpu.VMEM((2, F, CB), jnp.float32),
          pltpu.VMEM((NCAT * HT,), jnp.int32),
          pltpu.VMEM((NCAT * HT,), jnp.int32),
          pltpu.VMEM((NCAT * HT,), jnp.float32),
          pltpu.VMEM((NCAT * HT,), jnp.float32),
          pltpu.VMEM((HT,), jnp.float32),
          pltpu.VMEM((HT,), jnp.float32),
          pltpu.VMEM((8, CB), jnp.float32),
          pltpu.SemaphoreType.DMA,
          pltpu.SemaphoreType.DMA,
          pltpu.SemaphoreType.DMA,
          pltpu.SemaphoreType.DMA,
          pltpu.SemaphoreType.DMA,
      ],
  )
  out_t = run(x_t, *tbls, wcat, cvec)
  return out_t.T


# speedup vs baseline: 243.1768x; 1.1290x over previous
import functools

import jax
import jax.numpy as jnp
from jax import lax
from jax.experimental import pallas as pl
from jax.experimental.pallas import tpu as pltpu
from jax.experimental.pallas import tpu_sc as plsc

B, S, F = 16384, 200, 16
NCAT = 14
VOCAB = 100000
EDIM = 5

NC, NS = 2, 16
NW = NC * NS
BPT = B // NW
CB = 128
HT = 2 * CB
NCH = (S // 8) * (BPT // CB)
NHALF = NCH * 4

NVT = VOCAB // 128
VTAIL = VOCAB - NVT * 128
TPS = 49


def _kernel_body(x_hbm, t0_hbm, t1_hbm, t2_hbm, t3_hbm, t4_hbm,
                 wcat_hbm, cvec_hbm, out_hbm,
                 proj_sh, tblb, projb, wcat_v, cvec_v,
                 xb0, xb1, idxb0, idxb1, valb0, valb1, zb0, zb1, outb,
                 sx0, sx1, sg0, sg1, so):
  cid = lax.axis_index("c")
  sid = lax.axis_index("s")
  iota16 = lax.iota(jnp.int32, 16)
  iota14 = iota16 * NCAT
  tbl_hbms = [t0_hbm, t1_hbm, t2_hbm, t3_hbm, t4_hbm]

  pltpu.sync_copy(wcat_hbm, wcat_v)
  pltpu.sync_copy(cvec_hbm, cvec_v)

  def vtile(t, _):
    ds = [pltpu.async_copy(tbl_hbms[e].at[:, pl.ds(t * 128, 128)],
                           tblb.at[e], sx0) for e in range(EDIM)]
    for d in ds:
      d.wait()
    for c in range(NCAT):
      wc = [plsc.load_gather(wcat_v, [iota16 + (c * EDIM + e) * 16])
            for e in range(EDIM)]
      for g in range(8):
        acc = tblb[0, c, pl.ds(g * 16, 16)] * wc[0]
        for e in range(1, EDIM):
          acc = acc + tblb[e, c, pl.ds(g * 16, 16)] * wc[e]
        plsc.store_scatter(projb, [iota14 + (g * 16 * NCAT + c)], acc)
    pltpu.sync_copy(projb, proj_sh.at[pl.ds(t * (128 * NCAT), 128 * NCAT)])
    return 0

  ntiles = jnp.where(sid == NS - 1, NVT - (NS - 1) * TPS, TPS)
  lax.fori_loop(sid * TPS, sid * TPS + ntiles, vtile, 0)

  @pl.when(sid == NS - 1)
  def _():
    for e in range(EDIM):
      pltpu.sync_copy(tbl_hbms[e].at[:, pl.ds(NVT * 128, VTAIL)],
                      tblb.at[e, :, pl.ds(0, VTAIL)])

    def tgroup(g, _):
      q = g * 16 + iota16
      v = q // NCAT
      c = q % NCAT
      acc = jnp.zeros((16,), jnp.float32)
      for e in range(EDIM):
        tv = plsc.load_gather(tblb, [jnp.full((16,), e, jnp.int32), c, v])
        wv = plsc.load_gather(wcat_v, [(c * EDIM + e) * 16])
        acc = acc + tv * wv
      projb[pl.ds(g * 16, 16)] = acc
      return 0

    lax.fori_loop(0, VTAIL * NCAT // 16, tgroup, 0)
    pltpu.sync_copy(projb.at[pl.ds(0, VTAIL * NCAT)],
                    proj_sh.at[pl.ds(NVT * 128 * NCAT, VTAIL * NCAT)])

  plsc.subcore_barrier()

  wid = sid * NC + cid
  b_base = wid * BPT
  w0 = plsc.load_gather(cvec_v, [iota16])
  w1 = plsc.load_gather(cvec_v, [iota16 + 16])
  bvec = plsc.load_gather(cvec_v, [iota16 + 32])

  xbs, idxbs = [xb0, xb1], [idxb0, idxb1]
  valbs, zbs = [valb0, valb1], [zb0, zb1]
  sxs, sgs = [sx0, sx1], [sg0, sg1]

  def xslice(i):
    s0 = (i // 16) * 8 + (i % 4) * 2
    b0 = b_base + ((i // 4) % 4) * CB
    return x_hbm.at[pl.ds(s0, 2), :, pl.ds(b0, CB)]

  def build(i, p):
    xb, idxb, zb = xbs[p], idxbs[p], zbs[p]
    for r in range(2):
      for bg in range(8):
        base = r * CB + bg * 16
        for c in range(NCAT):
          v = xb[r, c + 2, pl.ds(bg * 16, 16)]
          idxb[pl.ds(c * HT + base, 16)] = v.astype(jnp.int32) * NCAT + c
        zb[pl.ds(base, 16)] = (xb[r, 0, pl.ds(bg * 16, 16)] * w0
                               + xb[r, 1, pl.ds(bg * 16, 16)] * w1 + bvec)

  def accum(pj, hp):
    valb, zb = valbs[pj], zbs[pj]
    pltpu.make_async_copy(proj_sh.at[idxbs[pj]], valb, sgs[pj]).wait()
    for r in range(2):
      for bg in range(8):
        base = r * CB + bg * 16
        z = zb[pl.ds(base, 16)]
        for c in range(NCAT):
          z = z + valb[pl.ds(c * HT + base, 16)]
        outb[2 * hp + r, pl.ds(bg * 16, 16)] = 1.0 / (1.0 + jnp.exp(-z))

  def outdma(c):
    return pltpu.make_async_copy(
        outb, out_hbm.at[pl.ds((c // 4) * 8, 8),
                         pl.ds(b_base + (c % 4) * CB, CB)], so)

  pltpu.async_copy(xslice(0), xbs[0], sxs[0])

  def chunk_loop(c, _):
    for h in range(4):
      p = h % 2
      i = c * 4 + h
      pltpu.make_async_copy(xslice(i), xbs[p], sxs[p]).wait()

      @pl.when(i < NHALF - 1)
      def _():
        pltpu.async_copy(xslice(i + 1), xbs[1 - p], sxs[1 - p])

      build(i, p)
      pltpu.async_copy(proj_sh.at[idxbs[p]], valbs[p], sgs[p])
      if h == 0:
        @pl.when(c > 0)
        def _():
          accum(1, 3)
          outdma(c - 1).start()
      elif h == 1:
        @pl.when(c > 0)
        def _():
          outdma(c - 1).wait()
        accum(0, 0)
      else:
        accum((h - 1) % 2, h - 1)
    return 0

  lax.fori_loop(0, NCH, chunk_loop, 0)
  accum(1, 3)
  d = outdma(NCH - 1)
  d.start()
  d.wait()


@jax.jit
def kernel(x, tables, W, b):
  x_t = x.transpose(1, 2, 0)
  tbls = [tables[:, :, e] for e in range(EDIM)]
  wcat = jnp.broadcast_to(W[2:, 0].reshape(NCAT, EDIM)[:, :, None],
                          (NCAT, EDIM, 16)).reshape(NCAT * EDIM * 16)
  cvec = jnp.broadcast_to(jnp.stack([W[0, 0], W[1, 0], b[0]])[:, None],
                          (3, 16)).reshape(3 * 16)

  mesh = plsc.VectorSubcoreMesh(core_axis_name="c", subcore_axis_name="s")
  run = pl.kernel(
      _kernel_body,
      out_type=jax.ShapeDtypeStruct((S, B), jnp.float32),
      mesh=mesh,
      compiler_params=pltpu.CompilerParams(
          needs_layout_passes=False, use_tc_tiling_on_sc=False),
      scratch_types=[
          pltpu.VMEM_SHARED((NCAT * VOCAB,), jnp.float32),
          pltpu.VMEM((EDIM, NCAT, 128), jnp.float32),
          pltpu.VMEM((128 * NCAT,), jnp.float32),
          pltpu.VMEM((NCAT * EDIM * 16,), jnp.float32),
          pltpu.VMEM((3 * 16,), jnp.float32),
          pltpu.VMEM((2, F, CB), jnp.float32),
          pltpu.VMEM((2, F, CB), jnp.float32),
          pltpu.VMEM((NCAT * HT,), jnp.int32),
          pltpu.VMEM((NCAT * HT,), jnp.int32),
          pltpu.VMEM((NCAT * HT,), jnp.float32),
          pltpu.VMEM((NCAT * HT,), jnp.float32),
          pltpu.VMEM((HT,), jnp.float32),
          pltpu.VMEM((HT,), jnp.float32),
          pltpu.VMEM((8, CB), jnp.float32),
          pltpu.SemaphoreType.DMA,
          pltpu.SemaphoreType.DMA,
          pltpu.SemaphoreType.DMA,
          pltpu.SemaphoreType.DMA,
          pltpu.SemaphoreType.DMA,
      ],
  )
  out_t = run(x_t, *tbls, wcat, cvec)
  return out_t.T


# speedup vs baseline: 293.6354x; 1.2075x over previous
import functools

import jax
import jax.numpy as jnp
from jax import lax
from jax.experimental import pallas as pl
from jax.experimental.pallas import tpu as pltpu
from jax.experimental.pallas import tpu_sc as plsc

B, S, F = 16384, 200, 16
NCAT = 14
VOCAB = 100000
EDIM = 5

NC, NS = 2, 16
NW = NC * NS
BPT = B // NW
CB = 128
HT = 2 * CB
NCH = (S // 8) * (BPT // CB)
NHALF = NCH * 4

NVT = VOCAB // 128
VTAIL = VOCAB - NVT * 128
TPS = 49


def _kernel_body(x_hbm, t0_hbm, t1_hbm, t2_hbm, t3_hbm, t4_hbm,
                 wcat_hbm, cvec_hbm, out_hbm,
                 proj_sh, tblb, projb, wcat_v, cvec_v,
                 xb0, xb1, idxb0, idxb1, valb0, valb1, zb0, zb1, outb,
                 sx0, sx1, sg0, sg1, sh0, sh1, so):
  cid = lax.axis_index("c")
  sid = lax.axis_index("s")
  iota16 = lax.iota(jnp.int32, 16)
  iota14 = iota16 * NCAT
  tbl_hbms = [t0_hbm, t1_hbm, t2_hbm, t3_hbm, t4_hbm]

  pltpu.sync_copy(wcat_hbm, wcat_v)
  pltpu.sync_copy(cvec_hbm, cvec_v)

  def vtile(t, _):
    ds = [pltpu.async_copy(tbl_hbms[e].at[:, pl.ds(t * 128, 128)],
                           tblb.at[e], sx0) for e in range(EDIM)]
    for d in ds:
      d.wait()
    for c in range(NCAT):
      wc = [plsc.load_gather(wcat_v, [iota16 + (c * EDIM + e) * 16])
            for e in range(EDIM)]
      for g in range(8):
        acc = tblb[0, c, pl.ds(g * 16, 16)] * wc[0]
        for e in range(1, EDIM):
          acc = acc + tblb[e, c, pl.ds(g * 16, 16)] * wc[e]
        plsc.store_scatter(projb, [iota14 + (g * 16 * NCAT + c)], acc)
    pltpu.sync_copy(projb, proj_sh.at[pl.ds(t * (128 * NCAT), 128 * NCAT)])
    return 0

  ntiles = jnp.where(sid == NS - 1, NVT - (NS - 1) * TPS, TPS)
  lax.fori_loop(sid * TPS, sid * TPS + ntiles, vtile, 0)

  @pl.when(sid == NS - 1)
  def _():
    for e in range(EDIM):
      pltpu.sync_copy(tbl_hbms[e].at[:, pl.ds(NVT * 128, VTAIL)],
                      tblb.at[e, :, pl.ds(0, VTAIL)])

    def tgroup(g, _):
      q = g * 16 + iota16
      v = q // NCAT
      c = q % NCAT
      acc = jnp.zeros((16,), jnp.float32)
      for e in range(EDIM):
        tv = plsc.load_gather(tblb, [jnp.full((16,), e, jnp.int32), c, v])
        wv = plsc.load_gather(wcat_v, [(c * EDIM + e) * 16])
        acc = acc + tv * wv
      projb[pl.ds(g * 16, 16)] = acc
      return 0

    lax.fori_loop(0, VTAIL * NCAT // 16, tgroup, 0)
    pltpu.sync_copy(projb.at[pl.ds(0, VTAIL * NCAT)],
                    proj_sh.at[pl.ds(NVT * 128 * NCAT, VTAIL * NCAT)])

  plsc.subcore_barrier()

  wid = sid * NC + cid
  b_base = wid * BPT
  w0 = plsc.load_gather(cvec_v, [iota16])
  w1 = plsc.load_gather(cvec_v, [iota16 + 16])
  bvec = plsc.load_gather(cvec_v, [iota16 + 32])

  xbs, idxbs = [xb0, xb1], [idxb0, idxb1]
  valbs, zbs = [valb0, valb1], [zb0, zb1]
  sxs, sgs, shs = [sx0, sx1], [sg0, sg1], [sh0, sh1]
  CSPLIT = 7 * HT

  def gather_a(p):
    return pltpu.make_async_copy(
        proj_sh.at[idxbs[p].at[pl.ds(0, CSPLIT)]],
        valbs[p].at[pl.ds(0, CSPLIT)], sgs[p])

  def gather_b(p):
    return pltpu.make_async_copy(
        proj_sh.at[idxbs[p].at[pl.ds(CSPLIT, NCAT * HT - CSPLIT)]],
        valbs[p].at[pl.ds(CSPLIT, NCAT * HT - CSPLIT)], shs[p])

  def xslice(i):
    s0 = (i // 16) * 8 + (i % 4) * 2
    bc = wid * (BPT // CB) + (i // 4) % 4
    return x_hbm.at[pl.ds(s0, 2), :, bc, :, :]

  def build_cols(p, c_lo, c_hi):
    xb, idxb = xbs[p], idxbs[p]
    for r in range(2):
      for bg in range(8):
        base = r * CB + bg * 16
        for c in range(c_lo, c_hi):
          v = xb[r, (c + 2) // 8, (c + 2) % 8, pl.ds(bg * 16, 16)]
          idxb[pl.ds(c * HT + base, 16)] = v.astype(jnp.int32) * NCAT + c

  def build_z(p):
    xb, zb = xbs[p], zbs[p]
    for r in range(2):
      for bg in range(8):
        base = r * CB + bg * 16
        zb[pl.ds(base, 16)] = (xb[r, 0, 0, pl.ds(bg * 16, 16)] * w0
                               + xb[r, 0, 1, pl.ds(bg * 16, 16)] * w1 + bvec)

  def accum(pj, hp):
    valb, zb = valbs[pj], zbs[pj]
    gather_a(pj).wait()
    zs = []
    for r in range(2):
      for bg in range(8):
        base = r * CB + bg * 16
        z = zb[pl.ds(base, 16)]
        for c in range(7):
          z = z + valb[pl.ds(c * HT + base, 16)]
        zs.append(z)
    gather_b(pj).wait()
    for r in range(2):
      for bg in range(8):
        base = r * CB + bg * 16
        z = zs[r * 8 + bg]
        for c in range(7, NCAT):
          z = z + valb[pl.ds(c * HT + base, 16)]
        outb[2 * hp + r, pl.ds(bg * 16, 16)] = 1.0 / (1.0 + jnp.exp(-z))

  def outdma(c):
    return pltpu.make_async_copy(
        outb, out_hbm.at[pl.ds((c // 4) * 8, 8),
                         pl.ds(b_base + (c % 4) * CB, CB)], so)

  pltpu.async_copy(xslice(0), xbs[0], sxs[0])

  def chunk_loop(c, _):
    for h in range(4):
      p = h % 2
      i = c * 4 + h
      pltpu.make_async_copy(xslice(i), xbs[p], sxs[p]).wait()

      @pl.when(i < NHALF - 1)
      def _():
        pltpu.async_copy(xslice(i + 1), xbs[1 - p], sxs[1 - p])

      build_cols(p, 0, 7)
      gather_a(p).start()
      build_cols(p, 7, NCAT)
      gather_b(p).start()
      build_z(p)
      if h == 0:
        @pl.when(c > 0)
        def _():
          accum(1, 3)
          outdma(c - 1).start()
      elif h == 1:
        @pl.when(c > 0)
        def _():
          outdma(c - 1).wait()
        accum(0, 0)
      else:
        accum((h - 1) % 2, h - 1)
    return 0

  lax.fori_loop(0, NCH, chunk_loop, 0)
  accum(1, 3)
  d = outdma(NCH - 1)
  d.start()
  d.wait()


@jax.jit
def kernel(x, tables, W, b):
  x_t = (x.transpose(1, 2, 0).reshape(S, 2, 8, B // CB, CB)
         .transpose(0, 1, 3, 2, 4))
  tbls = [tables[:, :, e] for e in range(EDIM)]
  wcat = jnp.broadcast_to(W[2:, 0].reshape(NCAT, EDIM)[:, :, None],
                          (NCAT, EDIM, 16)).reshape(NCAT * EDIM * 16)
  cvec = jnp.broadcast_to(jnp.stack([W[0, 0], W[1, 0], b[0]])[:, None],
                          (3, 16)).reshape(3 * 16)

  mesh = plsc.VectorSubcoreMesh(core_axis_name="c", subcore_axis_name="s")
  run = pl.kernel(
      _kernel_body,
      out_type=jax.ShapeDtypeStruct((S, B), jnp.float32),
      mesh=mesh,
      compiler_params=pltpu.CompilerParams(
          needs_layout_passes=False, use_tc_tiling_on_sc=False),
      scratch_types=[
          pltpu.VMEM_SHARED((NCAT * VOCAB,), jnp.float32),
          pltpu.VMEM((EDIM, NCAT, 128), jnp.float32),
          pltpu.VMEM((128 * NCAT,), jnp.float32),
          pltpu.VMEM((NCAT * EDIM * 16,), jnp.float32),
          pltpu.VMEM((3 * 16,), jnp.float32),
          pltpu.VMEM((2, 2, 8, CB), jnp.float32),
          pltpu.VMEM((2, 2, 8, CB), jnp.float32),
          pltpu.VMEM((NCAT * HT,), jnp.int32),
          pltpu.VMEM((NCAT * HT,), jnp.int32),
          pltpu.VMEM((NCAT * HT,), jnp.float32),
          pltpu.VMEM((NCAT * HT,), jnp.float32),
          pltpu.VMEM((HT,), jnp.float32),
          pltpu.VMEM((HT,), jnp.float32),
          pltpu.VMEM((8, CB), jnp.float32),
          pltpu.SemaphoreType.DMA,
          pltpu.SemaphoreType.DMA,
          pltpu.SemaphoreType.DMA,
          pltpu.SemaphoreType.DMA,
          pltpu.SemaphoreType.DMA,
          pltpu.SemaphoreType.DMA,
          pltpu.SemaphoreType.DMA,
      ],
  )
  out_t = run(x_t, *tbls, wcat, cvec)
  return out_t.T
